# Initial kernel scaffold; baseline (speedup 1.0000x reference)
import jax
import jax.numpy as jnp
from jax.experimental import pallas as pl


def kernel(z, edge_index_bottom, edge_index_mid, edge_index_full, idx_mid, idx_full, W_up1, b_up1, W_up2, b_up2, W_bot, b_bot, W0s, b0s, W01, b01, W02, b02, W1s, b1s, W11, b11, W12, b12, W_fin, b_fin, W_d1, b_d1, W_d2, b_d2, gamma, beta):
    raise NotImplementedError("write your pallas kernel here")


# speedup vs baseline: 4.5308x; 4.5308x over previous
import functools

import jax
import jax.numpy as jnp
from jax import lax
from jax.experimental import pallas as pl
from jax.experimental.pallas import tpu as pltpu
from jax.experimental.pallas import tpu_sc as plsc

N0, N1, N2 = 50000, 12500, 3125
NP0, NP1, NP2 = 51200, 13312, 4096
EP0, EP1, EP2 = 802816, 212992, 65536
LATENT = 32

_NC, _NS = 2, 16
_RZ = 32


def _leaky(x):
    return jnp.maximum(x, 0.01 * x)


def _sc_mesh():
    return plsc.VectorSubcoreMesh(
        core_axis_name="c", subcore_axis_name="s",
        num_cores=_NC, num_subcores=_NS)


@functools.lru_cache(maxsize=None)
def _make_sc_conv(npad, ep, hc, g):
    rows = ep // 128
    rows_w = rows // _NS
    n_chunks = rows_w // g
    rps = npad // _NS
    assert rows_w % g == 0 and rps % _RZ == 0 and hc % 16 == 0

    def body(a_hbm, b_hbm, dst_hbm, src_hbm, out_hbm,
             accum, dbuf, gd, gs, arows, brows, zbuf, lsem, gsem, ssem):
        c = lax.axis_index("c")
        s = lax.axis_index("s")

        zv = jnp.zeros((16,), jnp.float32)

        def zfill(i, _):
            for j in range(hc // 16):
                zbuf[i, pl.ds(j * 16, 16)] = zv
            return 0
        lax.fori_loop(0, _RZ, zfill, 0)

        def zacc(k, _):
            pltpu.sync_copy(zbuf, accum.at[pl.ds(s * rps + k * _RZ, _RZ)])
            return 0
        lax.fori_loop(0, rps // _RZ, zacc, 0)
        plsc.subcore_barrier()

        off = c * npad
        row0 = s * rows_w

        def chunk(t, _):
            base = row0 + t * g
            cp1 = pltpu.async_copy(dst_hbm.at[pl.ds(base, g)], dbuf, lsem)
            cp2 = pltpu.async_copy(src_hbm.at[pl.ds(base, g)], gs, lsem)
            cp1.wait()
            cp2.wait()
            for q in range(g):
                for j in range(8):
                    sl = pl.ds(j * 16, 16)
                    gd[q, sl] = dbuf[q, sl] + off
                    gs[q, sl] = gs[q, sl] + off
            cps = []
            for q in range(g):
                cps.append(pltpu.async_copy(
                    a_hbm.at[gd.at[q]], arows.at[pl.ds(q * 128, 128)], gsem))
                cps.append(pltpu.async_copy(
                    b_hbm.at[gs.at[q]], brows.at[pl.ds(q * 128, 128)], gsem))
            for cp in cps:
                cp.wait()

            def leak(r, _):
                for u in range(4):
                    for j in range(hc // 16):
                        sl = pl.ds(j * 16, 16)
                        m = arows[r * 4 + u, sl] + brows[r * 4 + u, sl]
                        arows[r * 4 + u, sl] = jnp.maximum(m, m * 0.01)
                return 0
            lax.fori_loop(0, g * 32, leak, 0)

            cps2 = []
            for q in range(g):
                cps2.append(pltpu.async_copy(
                    arows.at[pl.ds(q * 128, 128)],
                    accum.at[dbuf.at[q]], ssem, add=True))
            for cp in cps2:
                cp.wait()
            return 0
        lax.fori_loop(0, n_chunks, chunk, 0)
        plsc.subcore_barrier()

        def cout(k, _):
            r = s * rps + k * _RZ
            pltpu.sync_copy(accum.at[pl.ds(r, _RZ)], zbuf)
            pltpu.sync_copy(zbuf, out_hbm.at[pl.ds(off + r, _RZ)])
            return 0
        lax.fori_loop(0, rps // _RZ, cout, 0)

    return pl.kernel(
        body,
        out_type=jax.ShapeDtypeStruct((2 * npad, hc), jnp.float32),
        mesh=_sc_mesh(),
        compiler_params=pltpu.CompilerParams(use_tc_tiling_on_sc=False),
        scratch_types=[
            pltpu.VMEM_SHARED((npad, hc), jnp.float32),
            pltpu.VMEM((g, 128), jnp.int32),
            pltpu.VMEM((g, 128), jnp.int32),
            pltpu.VMEM((g, 128), jnp.int32),
            pltpu.VMEM((g * 128, hc), jnp.float32),
            pltpu.VMEM((g * 128, hc), jnp.float32),
            pltpu.VMEM((_RZ, hc), jnp.float32),
            pltpu.SemaphoreType.DMA,
            pltpu.SemaphoreType.DMA,
            pltpu.SemaphoreType.DMA,
        ],
    )


@functools.lru_cache(maxsize=None)
def _make_sc_count(npad, ep, g):
    rows = ep // 128
    rows_w = rows // (_NC * _NS)
    n_chunks = rows_w // g
    rps = npad // _NS
    assert rows_w % g == 0 and rps % _RZ == 0

    def body(dst_hbm, out_hbm, accum, dbuf, ones, zbuf, lsem, ssem):
        c = lax.axis_index("c")
        s = lax.axis_index("s")

        zv = jnp.zeros((16,), jnp.float32)
        ov = jnp.ones((16,), jnp.float32)

        def zfill(i, _):
            zbuf[i, pl.ds(0, 16)] = zv
            return 0
        lax.fori_loop(0, _RZ, zfill, 0)

        def ofill(i, _):
            ones[i, pl.ds(0, 16)] = ov
            return 0
        lax.fori_loop(0, 128, ofill, 0)

        def zacc(k, _):
            pltpu.sync_copy(zbuf, accum.at[pl.ds(s * rps + k * _RZ, _RZ)])
            return 0
        lax.fori_loop(0, rps // _RZ, zacc, 0)
        plsc.subcore_barrier()

        row0 = (c * _NS + s) * rows_w

        def chunk(t, _):
            pltpu.async_copy(
                dst_hbm.at[pl.ds(row0 + t * g, g)], dbuf, lsem).wait()
            cps = []
            for q in range(g):
                cps.append(pltpu.async_copy(
                    ones, accum.at[dbuf.at[q]], ssem, add=True))
            for cp in cps:
                cp.wait()
            return 0
        lax.fori_loop(0, n_chunks, chunk, 0)
        plsc.subcore_barrier()

        def cout(k, _):
            r = s * rps + k * _RZ
            pltpu.sync_copy(accum.at[pl.ds(r, _RZ)], zbuf)
            pltpu.sync_copy(zbuf, out_hbm.at[pl.ds(c * npad + r, _RZ)])
            return 0
        lax.fori_loop(0, rps // _RZ, cout, 0)

    return pl.kernel(
        body,
        out_type=jax.ShapeDtypeStruct((2 * npad, 16), jnp.float32),
        mesh=_sc_mesh(),
        compiler_params=pltpu.CompilerParams(use_tc_tiling_on_sc=False),
        scratch_types=[
            pltpu.VMEM_SHARED((npad, 16), jnp.float32),
            pltpu.VMEM((g, 128), jnp.int32),
            pltpu.VMEM((128, 16), jnp.float32),
            pltpu.VMEM((_RZ, 16), jnp.float32),
            pltpu.SemaphoreType.DMA,
            pltpu.SemaphoreType.DMA,
        ],
    )


def _tc_tables(sources, cnt, w, b, *, np_src, n_valid, np_out, f, h, act):
    hc = h // 2
    hw = f // 2
    r = 1024
    nb = np_out // r
    nbs = np_src // r
    n_src = len(sources)
    wt = jnp.transpose(w.reshape(2 * f, 2, hc), (1, 0, 2))
    bt = b.reshape(2, 1, hc)

    def body(*refs):
        s_refs = refs[:2 * n_src]
        cl_ref, ch_ref, w_ref, b_ref, a_ref, b_out_ref = refs[2 * n_src:]
        i = pl.program_id(1)
        xa = s_refs[0][...]
        xb = s_refs[1][...]
        for k in range(1, n_src):
            xa = xa + s_refs[2 * k][...]
            xb = xb + s_refs[2 * k + 1][...]
        cntv = jnp.maximum(cl_ref[:, 0] + ch_ref[:, 0], 1.0)[:, None]
        xa = xa / cntv
        xb = xb / cntv
        if act:
            xa = _leaky(xa)
            xb = _leaky(xb)
        gid = i * r + lax.broadcasted_iota(jnp.int32, (r, 1), 0)
        valid = gid < n_valid
        xa = jnp.where(valid, xa, 0.0)
        xb = jnp.where(valid, xb, 0.0)
        wc = w_ref[0]
        wd = wc[:f, :] - wc[f:, :]
        wj = wc[f:, :]
        a_ref[...] = (xa @ wd[:hw, :] + xb @ wd[hw:, :]) + b_ref[0]
        b_out_ref[...] = xa @ wj[:hw, :] + xb @ wj[hw:, :]

    hcs = sources[0].shape[1]
    src_lo = pl.BlockSpec(
        (r, hcs), lambda c, i: (jnp.minimum(i, nbs - 1), 0))
    src_hi = pl.BlockSpec(
        (r, hcs), lambda c, i: (nbs + jnp.minimum(i, nbs - 1), 0))
    in_specs = []
    operands = []
    for s_arr in sources:
        in_specs += [src_lo, src_hi]
        operands += [s_arr, s_arr]
    in_specs += [
        pl.BlockSpec((r, 16), lambda c, i: (jnp.minimum(i, nbs - 1), 0)),
        pl.BlockSpec((r, 16), lambda c, i: (nbs + jnp.minimum(i, nbs - 1), 0)),
        pl.BlockSpec((1, 2 * f, hc), lambda c, i: (c, 0, 0)),
        pl.BlockSpec((1, 1, hc), lambda c, i: (c, 0, 0)),
    ]
    operands += [cnt, cnt, wt, bt]
    out_spec = pl.BlockSpec((r, hc), lambda c, i: (c * nb + i, 0))
    return pl.pallas_call(
        body,
        grid=(2, nb),
        in_specs=in_specs,
        out_specs=[out_spec, out_spec],
        out_shape=[jax.ShapeDtypeStruct((2 * np_out, hc), jnp.float32)] * 2,
    )(*operands)


def _tc_latent(z, w_up1, b_up1, w_up2p, b_up2p, w_bot, b_bot):
    r = 1024
    nb = NP2 // r
    hc = 128

    def body(z_ref, wu1_ref, bu1_ref, wu2_ref, bu2_ref, wb_ref, bb_ref,
             a_ref, b_out_ref):
        z2 = z_ref[0]
        hh = _leaky(z2 @ wu1_ref[...] + bu1_ref[...][None, :])
        x0 = lax.dot_general(
            wu2_ref[...], hh, (((0,), (1,)), ((), ())),
            preferred_element_type=jnp.float32)
        x0 = x0 + bu2_ref[...][:, None]
        wd = wb_ref[:LATENT, :] - wb_ref[LATENT:, :]
        wj = wb_ref[LATENT:, :]
        a_ref[...] = x0 @ wd + bb_ref[...][None, :]
        b_out_ref[...] = x0 @ wj

    out_spec = pl.BlockSpec((r, hc), lambda c, i: (c * nb + i, 0))
    return pl.pallas_call(
        body,
        grid=(2, nb),
        in_specs=[
            pl.BlockSpec((1, LATENT, 1), lambda c, i: (0, 0, 0)),
            pl.BlockSpec((1, 64), lambda c, i: (0, 0)),
            pl.BlockSpec((64,), lambda c, i: (0,)),
            pl.BlockSpec((64, r), lambda c, i: (0, i)),
            pl.BlockSpec((r,), lambda c, i: (i,)),
            pl.BlockSpec((2 * LATENT, hc), lambda c, i: (0, c)),
            pl.BlockSpec((hc,), lambda c, i: (c,)),
        ],
        out_specs=[out_spec, out_spec],
        out_shape=[jax.ShapeDtypeStruct((2 * NP2, hc), jnp.float32)] * 2,
    )(z, w_up1, b_up1, w_up2p, b_up2p, w_bot, b_bot)


def _tc_decoder(s_fin, cnt, w_d1, b_d1, w_d2, b_d2, gamma, beta):
    r = 1024
    nb = NP0 // r

    def body(sl_ref, sh_ref, cl_ref, ch_ref, w1_ref, b1_ref, w2_ref, b2_ref,
             g_ref, be_ref, o_ref):
        cntv = jnp.maximum(cl_ref[:, 0] + ch_ref[:, 0], 1.0)[:, None]
        xa = sl_ref[...] / cntv
        xb = sh_ref[...] / cntv
        hh = _leaky(xa @ w1_ref[:32, :] + xb @ w1_ref[32:, :]
                    + b1_ref[...][None, :])
        o = hh @ w2_ref[...] + b2_ref[...][None, :]
        mu = jnp.mean(o, axis=-1, keepdims=True)
        d = o - mu
        var = jnp.mean(d * d, axis=-1, keepdims=True)
        o_ref[...] = (d / jnp.sqrt(var + 1e-5)) * g_ref[...][None, :] \
            + be_ref[...][None, :]

    return pl.pallas_call(
        body,
        grid=(nb,),
        in_specs=[
            pl.BlockSpec((r, 32), lambda i: (i, 0)),
            pl.BlockSpec((r, 32), lambda i: (nb + i, 0)),
            pl.BlockSpec((r, 16), lambda i: (i, 0)),
            pl.BlockSpec((r, 16), lambda i: (nb + i, 0)),
            pl.BlockSpec((64, 32), lambda i: (0, 0)),
            pl.BlockSpec((32,), lambda i: (0,)),
            pl.BlockSpec((32, 3), lambda i: (0, 0)),
            pl.BlockSpec((3,), lambda i: (0,)),
            pl.BlockSpec((3,), lambda i: (0,)),
            pl.BlockSpec((3,), lambda i: (0,)),
        ],
        out_specs=pl.BlockSpec((r, 3), lambda i: (i, 0)),
        out_shape=jax.ShapeDtypeStruct((NP0, 3), jnp.float32),
    )(s_fin, s_fin, cnt, cnt, w_d1, b_d1, w_d2, b_d2, gamma, beta)


def _prep_edges(edge_index, npad, ep):
    src = edge_index[0]
    dst = edge_index[1]
    pad = ep - src.shape[0]
    dummy = jnp.full((pad,), npad - 1, jnp.int32)
    src = jnp.concatenate([src, dummy]).reshape(ep // 128, 128)
    dst = jnp.concatenate([dst, dummy]).reshape(ep // 128, 128)
    return src, dst


def kernel(z, edge_index_bottom, edge_index_mid, edge_index_full, idx_mid,
           idx_full, W_up1, b_up1, W_up2, b_up2, W_bot, b_bot, W0s, b0s, W01,
           b01, W02, b02, W1s, b1s, W11, b11, W12, b12, W_fin, b_fin, W_d1,
           b_d1, W_d2, b_d2, gamma, beta):
    del idx_mid, idx_full
    sb2, db2 = _prep_edges(edge_index_bottom, NP2, EP2)
    sb1, db1 = _prep_edges(edge_index_mid, NP1, EP1)
    sb0, db0 = _prep_edges(edge_index_full, NP0, EP0)

    cnt_b = _make_sc_count(NP2, EP2, 4)(db2)
    cnt_m = _make_sc_count(NP1, EP1, 4)(db1)
    cnt_f = _make_sc_count(NP0, EP0, 4)(db0)

    w_up2p = jnp.pad(W_up2, ((0, 0), (0, NP2 - N2)))
    b_up2p = jnp.pad(b_up2, (0, NP2 - N2))

    a1, b1 = _tc_latent(z, W_up1, b_up1, w_up2p, b_up2p, W_bot, b_bot)
    s1 = _make_sc_conv(NP2, EP2, 128, 2)(a1, b1, db2, sb2)

    a0s, b0s_t = _tc_tables([s1], cnt_b, W0s, b0s, np_src=NP2, n_valid=N2,
                            np_out=NP1, f=256, h=128, act=False)
    s_skip = _make_sc_conv(NP1, EP1, 64, 2)(a0s, b0s_t, db1, sb1)

    a01, b01_t = _tc_tables([s1], cnt_b, W01, b01, np_src=NP2, n_valid=N2,
                            np_out=NP2, f=256, h=64, act=False)
    s_h = _make_sc_conv(NP2, EP2, 32, 4)(a01, b01_t, db2, sb2)

    a02, b02_t = _tc_tables([s_h], cnt_b, W02, b02, np_src=NP2, n_valid=N2,
                            np_out=NP1, f=64, h=128, act=False)
    s_02 = _make_sc_conv(NP1, EP1, 64, 2)(a02, b02_t, db1, sb1)

    a1s, b1s_t = _tc_tables([s_skip, s_02], cnt_m, W1s, b1s, np_src=NP1,
                            n_valid=N1, np_out=NP0, f=128, h=64, act=True)
    s_1s = _make_sc_conv(NP0, EP0, 32, 2)(a1s, b1s_t, db0, sb0)

    a11, b11_t = _tc_tables([s_skip, s_02], cnt_m, W11, b11, np_src=NP1,
                            n_valid=N1, np_out=NP1, f=128, h=32, act=True)
    s_11 = _make_sc_conv(NP1, EP1, 16, 8)(a11, b11_t, db1, sb1)

    a12, b12_t = _tc_tables([s_11], cnt_m, W12, b12, np_src=NP1, n_valid=N1,
                            np_out=NP0, f=32, h=64, act=False)
    s_12 = _make_sc_conv(NP0, EP0, 32, 2)(a12, b12_t, db0, sb0)

    afin, bfin_t = _tc_tables([s_1s, s_12], cnt_f, W_fin, b_fin, np_src=NP0,
                              n_valid=N0, np_out=NP0, f=64, h=64, act=True)
    s_fin = _make_sc_conv(NP0, EP0, 32, 2)(afin, bfin_t, db0, sb0)

    out = _tc_decoder(s_fin, cnt_f, W_d1, b_d1, W_d2, b_d2, gamma, beta)
    return out[:N0]


# speedup vs baseline: 4.5486x; 1.0039x over previous
import functools

import jax
import jax.numpy as jnp
from jax import lax
from jax.experimental import pallas as pl
from jax.experimental.pallas import tpu as pltpu
from jax.experimental.pallas import tpu_sc as plsc

N0, N1, N2 = 50000, 12500, 3125
NP0, NP1, NP2 = 51200, 13312, 4096
EP0, EP1, EP2 = 802816, 212992, 65536
LATENT = 32

_NC, _NS = 2, 16
_RZ = 32


def _leaky(x):
    return jnp.maximum(x, 0.01 * x)


def _sc_mesh():
    return plsc.VectorSubcoreMesh(
        core_axis_name="c", subcore_axis_name="s",
        num_cores=_NC, num_subcores=_NS)


@functools.lru_cache(maxsize=None)
def _make_sc_conv(npad, ep, hc, g):
    rows = ep // 128
    rows_w = rows // _NS
    n_chunks = rows_w // g
    rps = npad // _NS
    assert rows_w % g == 0 and rps % _RZ == 0 and hc % 16 == 0

    assert n_chunks % 2 == 0 and n_chunks >= 4

    def body(a_hbm, b_hbm, dst_hbm, src_hbm, out_hbm,
             accum, dbuf, sbuf, gd, gs, arows, brows, zbuf,
             lsem, gsemA, gsemB, ssem, zsem):
        c = lax.axis_index("c")
        s = lax.axis_index("s")
        gsems = (gsemA, gsemB)

        zv = jnp.zeros((16,), jnp.float32)

        def zfill(i, _):
            for j in range(hc // 16):
                zbuf[i, pl.ds(j * 16, 16)] = zv
            return 0
        lax.fori_loop(0, _RZ, zfill, 0)

        def zacc(k, _):
            pltpu.async_copy(zbuf, accum.at[pl.ds(s * rps + k * _RZ, _RZ)],
                             zsem)
            return 0
        lax.fori_loop(0, rps // _RZ, zacc, 0)

        def zdrain(k, _):
            pltpu.make_async_copy(
                zbuf, accum.at[pl.ds(s * rps + k * _RZ, _RZ)], zsem).wait()
            return 0
        lax.fori_loop(0, rps // _RZ, zdrain, 0)
        plsc.subcore_barrier()

        off = c * npad
        row0 = s * rows_w

        def idx_fire(t, sl):
            base = row0 + t * g
            pltpu.async_copy(dst_hbm.at[pl.ds(base, g)], dbuf.at[sl], lsem)
            pltpu.async_copy(src_hbm.at[pl.ds(base, g)], sbuf.at[sl], lsem)

        def idx_wait(t, sl):
            base = row0 + t * g
            pltpu.make_async_copy(
                dst_hbm.at[pl.ds(base, g)], dbuf.at[sl], lsem).wait()
            pltpu.make_async_copy(
                src_hbm.at[pl.ds(base, g)], sbuf.at[sl], lsem).wait()

        def shift_fire_gather(sl):
            for q in range(g):
                for j in range(8):
                    s16 = pl.ds(j * 16, 16)
                    gd[sl, q, s16] = dbuf[sl, q, s16] + off
                    gs[sl, q, s16] = sbuf[sl, q, s16] + off
            for q in range(g):
                pltpu.async_copy(
                    a_hbm.at[gd.at[sl, q]],
                    arows.at[sl, pl.ds(q * 128, 128)], gsems[sl])
                pltpu.async_copy(
                    b_hbm.at[gs.at[sl, q]],
                    brows.at[sl, pl.ds(q * 128, 128)], gsems[sl])

        def gather_wait(sl):
            for q in range(g):
                pltpu.make_async_copy(
                    a_hbm.at[gd.at[sl, q]],
                    arows.at[sl, pl.ds(q * 128, 128)], gsems[sl]).wait()
                pltpu.make_async_copy(
                    b_hbm.at[gs.at[sl, q]],
                    brows.at[sl, pl.ds(q * 128, 128)], gsems[sl]).wait()

        def leak(sl):
            def lk(r, _):
                for u in range(4):
                    for j in range(hc // 16):
                        s16 = pl.ds(j * 16, 16)
                        m = (arows[sl, r * 4 + u, s16]
                             + brows[sl, r * 4 + u, s16])
                        arows[sl, r * 4 + u, s16] = jnp.maximum(m, m * 0.01)
                return 0
            lax.fori_loop(0, g * 32, lk, 0)

        def scatter(sl):
            cps = []
            for q in range(g):
                cps.append(pltpu.async_copy(
                    arows.at[sl, pl.ds(q * 128, 128)],
                    accum.at[dbuf.at[sl, q]], ssem, add=True))
            for cp in cps:
                cp.wait()

        idx_fire(0, 0)
        idx_wait(0, 0)
        shift_fire_gather(0)
        idx_fire(1, 1)

        def pairstep(tt, _):
            for ph in range(2):
                t = 2 * tt + ph
                sl = ph
                osl = 1 - ph
                gather_wait(sl)
                leak(sl)
                scatter(sl)

                @pl.when(t + 2 < n_chunks)
                def _():
                    idx_fire(t + 2, sl)

                @pl.when(t + 1 < n_chunks)
                def _():
                    idx_wait(t + 1, osl)
                    shift_fire_gather(osl)
            return 0
        lax.fori_loop(0, n_chunks // 2, pairstep, 0)
        plsc.subcore_barrier()

        pltpu.sync_copy(accum.at[pl.ds(s * rps, rps)],
                        out_hbm.at[pl.ds(off + s * rps, rps)])

    return pl.kernel(
        body,
        out_type=jax.ShapeDtypeStruct((2 * npad, hc), jnp.float32),
        mesh=_sc_mesh(),
        compiler_params=pltpu.CompilerParams(use_tc_tiling_on_sc=False),
        scratch_types=[
            pltpu.VMEM_SHARED((npad, hc), jnp.float32),
            pltpu.VMEM((2, g, 128), jnp.int32),
            pltpu.VMEM((2, g, 128), jnp.int32),
            pltpu.VMEM((2, g, 128), jnp.int32),
            pltpu.VMEM((2, g, 128), jnp.int32),
            pltpu.VMEM((2, g * 128, hc), jnp.float32),
            pltpu.VMEM((2, g * 128, hc), jnp.float32),
            pltpu.VMEM((_RZ, hc), jnp.float32),
            pltpu.SemaphoreType.DMA,
            pltpu.SemaphoreType.DMA,
            pltpu.SemaphoreType.DMA,
            pltpu.SemaphoreType.DMA,
            pltpu.SemaphoreType.DMA,
        ],
    )


@functools.lru_cache(maxsize=None)
def _make_sc_count(npad, ep, g):
    rows = ep // 128
    rows_w = rows // (_NC * _NS)
    n_chunks = rows_w // g
    rps = npad // _NS
    assert rows_w % g == 0 and rps % _RZ == 0

    def body(dst_hbm, out_hbm, accum, dbuf, ones, zbuf, lsem, ssem):
        c = lax.axis_index("c")
        s = lax.axis_index("s")

        zv = jnp.zeros((16,), jnp.float32)
        ov = jnp.ones((16,), jnp.float32)

        def zfill(i, _):
            zbuf[i, pl.ds(0, 16)] = zv
            return 0
        lax.fori_loop(0, _RZ, zfill, 0)

        def ofill(i, _):
            ones[i, pl.ds(0, 16)] = ov
            return 0
        lax.fori_loop(0, 128, ofill, 0)

        def zacc(k, _):
            pltpu.async_copy(zbuf, accum.at[pl.ds(s * rps + k * _RZ, _RZ)],
                             ssem)
            return 0
        lax.fori_loop(0, rps // _RZ, zacc, 0)

        def zdrain(k, _):
            pltpu.make_async_copy(
                zbuf, accum.at[pl.ds(s * rps + k * _RZ, _RZ)], ssem).wait()
            return 0
        lax.fori_loop(0, rps // _RZ, zdrain, 0)
        plsc.subcore_barrier()

        row0 = (c * _NS + s) * rows_w

        def chunk(t, _):
            pltpu.async_copy(
                dst_hbm.at[pl.ds(row0 + t * g, g)], dbuf, lsem).wait()
            cps = []
            for q in range(g):
                cps.append(pltpu.async_copy(
                    ones, accum.at[dbuf.at[q]], ssem, add=True))
            for cp in cps:
                cp.wait()
            return 0
        lax.fori_loop(0, n_chunks, chunk, 0)
        plsc.subcore_barrier()

        pltpu.sync_copy(accum.at[pl.ds(s * rps, rps)],
                        out_hbm.at[pl.ds(c * npad + s * rps, rps)])

    return pl.kernel(
        body,
        out_type=jax.ShapeDtypeStruct((2 * npad, 16), jnp.float32),
        mesh=_sc_mesh(),
        compiler_params=pltpu.CompilerParams(use_tc_tiling_on_sc=False),
        scratch_types=[
            pltpu.VMEM_SHARED((npad, 16), jnp.float32),
            pltpu.VMEM((g, 128), jnp.int32),
            pltpu.VMEM((128, 16), jnp.float32),
            pltpu.VMEM((_RZ, 16), jnp.float32),
            pltpu.SemaphoreType.DMA,
            pltpu.SemaphoreType.DMA,
        ],
    )


def _tc_tables(sources, cnt, w, b, *, np_src, n_valid, np_out, f, h, act):
    hc = h // 2
    hw = f // 2
    r = 1024
    nb = np_out // r
    nbs = np_src // r
    n_src = len(sources)
    wt = jnp.transpose(w.reshape(2 * f, 2, hc), (1, 0, 2))
    bt = b.reshape(2, 1, hc)

    def body(*refs):
        s_refs = refs[:2 * n_src]
        cl_ref, ch_ref, w_ref, b_ref, a_ref, b_out_ref = refs[2 * n_src:]
        i = pl.program_id(1)
        xa = s_refs[0][...]
        xb = s_refs[1][...]
        for k in range(1, n_src):
            xa = xa + s_refs[2 * k][...]
            xb = xb + s_refs[2 * k + 1][...]
        cntv = jnp.maximum(cl_ref[:, 0] + ch_ref[:, 0], 1.0)[:, None]
        xa = xa / cntv
        xb = xb / cntv
        if act:
            xa = _leaky(xa)
            xb = _leaky(xb)
        gid = i * r + lax.broadcasted_iota(jnp.int32, (r, 1), 0)
        valid = gid < n_valid
        xa = jnp.where(valid, xa, 0.0)
        xb = jnp.where(valid, xb, 0.0)
        wc = w_ref[0]
        wd = wc[:f, :] - wc[f:, :]
        wj = wc[f:, :]
        a_ref[...] = (xa @ wd[:hw, :] + xb @ wd[hw:, :]) + b_ref[0]
        b_out_ref[...] = xa @ wj[:hw, :] + xb @ wj[hw:, :]

    hcs = sources[0].shape[1]
    src_lo = pl.BlockSpec(
        (r, hcs), lambda c, i: (jnp.minimum(i, nbs - 1), 0))
    src_hi = pl.BlockSpec(
        (r, hcs), lambda c, i: (nbs + jnp.minimum(i, nbs - 1), 0))
    in_specs = []
    operands = []
    for s_arr in sources:
        in_specs += [src_lo, src_hi]
        operands += [s_arr, s_arr]
    in_specs += [
        pl.BlockSpec((r, 16), lambda c, i: (jnp.minimum(i, nbs - 1), 0)),
        pl.BlockSpec((r, 16), lambda c, i: (nbs + jnp.minimum(i, nbs - 1), 0)),
        pl.BlockSpec((1, 2 * f, hc), lambda c, i: (c, 0, 0)),
        pl.BlockSpec((1, 1, hc), lambda c, i: (c, 0, 0)),
    ]
    operands += [cnt, cnt, wt, bt]
    out_spec = pl.BlockSpec((r, hc), lambda c, i: (c * nb + i, 0))
    return pl.pallas_call(
        body,
        grid=(2, nb),
        in_specs=in_specs,
        out_specs=[out_spec, out_spec],
        out_shape=[jax.ShapeDtypeStruct((2 * np_out, hc), jnp.float32)] * 2,
    )(*operands)


def _tc_latent(z, w_up1, b_up1, w_up2p, b_up2p, w_bot, b_bot):
    r = 1024
    nb = NP2 // r
    hc = 128

    def body(z_ref, wu1_ref, bu1_ref, wu2_ref, bu2_ref, wb_ref, bb_ref,
             a_ref, b_out_ref):
        z2 = z_ref[0]
        hh = _leaky(z2 @ wu1_ref[...] + bu1_ref[...][None, :])
        x0 = lax.dot_general(
            wu2_ref[...], hh, (((0,), (1,)), ((), ())),
            preferred_element_type=jnp.float32)
        x0 = x0 + bu2_ref[...][:, None]
        wd = wb_ref[:LATENT, :] - wb_ref[LATENT:, :]
        wj = wb_ref[LATENT:, :]
        a_ref[...] = x0 @ wd + bb_ref[...][None, :]
        b_out_ref[...] = x0 @ wj

    out_spec = pl.BlockSpec((r, hc), lambda c, i: (c * nb + i, 0))
    return pl.pallas_call(
        body,
        grid=(2, nb),
        in_specs=[
            pl.BlockSpec((1, LATENT, 1), lambda c, i: (0, 0, 0)),
            pl.BlockSpec((1, 64), lambda c, i: (0, 0)),
            pl.BlockSpec((64,), lambda c, i: (0,)),
            pl.BlockSpec((64, r), lambda c, i: (0, i)),
            pl.BlockSpec((r,), lambda c, i: (i,)),
            pl.BlockSpec((2 * LATENT, hc), lambda c, i: (0, c)),
            pl.BlockSpec((hc,), lambda c, i: (c,)),
        ],
        out_specs=[out_spec, out_spec],
        out_shape=[jax.ShapeDtypeStruct((2 * NP2, hc), jnp.float32)] * 2,
    )(z, w_up1, b_up1, w_up2p, b_up2p, w_bot, b_bot)


def _tc_decoder(s_fin, cnt, w_d1, b_d1, w_d2, b_d2, gamma, beta):
    r = 1024
    nb = NP0 // r

    def body(sl_ref, sh_ref, cl_ref, ch_ref, w1_ref, b1_ref, w2_ref, b2_ref,
             g_ref, be_ref, o_ref):
        cntv = jnp.maximum(cl_ref[:, 0] + ch_ref[:, 0], 1.0)[:, None]
        xa = sl_ref[...] / cntv
        xb = sh_ref[...] / cntv
        hh = _leaky(xa @ w1_ref[:32, :] + xb @ w1_ref[32:, :]
                    + b1_ref[...][None, :])
        o = hh @ w2_ref[...] + b2_ref[...][None, :]
        mu = jnp.mean(o, axis=-1, keepdims=True)
        d = o - mu
        var = jnp.mean(d * d, axis=-1, keepdims=True)
        o_ref[...] = (d / jnp.sqrt(var + 1e-5)) * g_ref[...][None, :] \
            + be_ref[...][None, :]

    return pl.pallas_call(
        body,
        grid=(nb,),
        in_specs=[
            pl.BlockSpec((r, 32), lambda i: (i, 0)),
            pl.BlockSpec((r, 32), lambda i: (nb + i, 0)),
            pl.BlockSpec((r, 16), lambda i: (i, 0)),
            pl.BlockSpec((r, 16), lambda i: (nb + i, 0)),
            pl.BlockSpec((64, 32), lambda i: (0, 0)),
            pl.BlockSpec((32,), lambda i: (0,)),
            pl.BlockSpec((32, 3), lambda i: (0, 0)),
            pl.BlockSpec((3,), lambda i: (0,)),
            pl.BlockSpec((3,), lambda i: (0,)),
            pl.BlockSpec((3,), lambda i: (0,)),
        ],
        out_specs=pl.BlockSpec((r, 3), lambda i: (i, 0)),
        out_shape=jax.ShapeDtypeStruct((NP0, 3), jnp.float32),
    )(s_fin, s_fin, cnt, cnt, w_d1, b_d1, w_d2, b_d2, gamma, beta)


def _prep_edges(edge_index, npad, ep):
    src = edge_index[0]
    dst = edge_index[1]
    pad = ep - src.shape[0]
    dummy = jnp.full((pad,), npad - 1, jnp.int32)
    src = jnp.concatenate([src, dummy]).reshape(ep // 128, 128)
    dst = jnp.concatenate([dst, dummy]).reshape(ep // 128, 128)
    return src, dst


def kernel(z, edge_index_bottom, edge_index_mid, edge_index_full, idx_mid,
           idx_full, W_up1, b_up1, W_up2, b_up2, W_bot, b_bot, W0s, b0s, W01,
           b01, W02, b02, W1s, b1s, W11, b11, W12, b12, W_fin, b_fin, W_d1,
           b_d1, W_d2, b_d2, gamma, beta):
    del idx_mid, idx_full
    sb2, db2 = _prep_edges(edge_index_bottom, NP2, EP2)
    sb1, db1 = _prep_edges(edge_index_mid, NP1, EP1)
    sb0, db0 = _prep_edges(edge_index_full, NP0, EP0)

    cnt_b = _make_sc_count(NP2, EP2, 2)(db2)
    cnt_m = _make_sc_count(NP1, EP1, 2)(db1)
    cnt_f = _make_sc_count(NP0, EP0, 14)(db0)

    w_up2p = jnp.pad(W_up2, ((0, 0), (0, NP2 - N2)))
    b_up2p = jnp.pad(b_up2, (0, NP2 - N2))

    a1, b1 = _tc_latent(z, W_up1, b_up1, w_up2p, b_up2p, W_bot, b_bot)
    s1 = _make_sc_conv(NP2, EP2, 128, 1)(a1, b1, db2, sb2)

    a0s, b0s_t = _tc_tables([s1], cnt_b, W0s, b0s, np_src=NP2, n_valid=N2,
                            np_out=NP1, f=256, h=128, act=False)
    s_skip = _make_sc_conv(NP1, EP1, 64, 1)(a0s, b0s_t, db1, sb1)

    a01, b01_t = _tc_tables([s1], cnt_b, W01, b01, np_src=NP2, n_valid=N2,
                            np_out=NP2, f=256, h=64, act=False)
    s_h = _make_sc_conv(NP2, EP2, 32, 2)(a01, b01_t, db2, sb2)

    a02, b02_t = _tc_tables([s_h], cnt_b, W02, b02, np_src=NP2, n_valid=N2,
                            np_out=NP1, f=64, h=128, act=False)
    s_02 = _make_sc_conv(NP1, EP1, 64, 1)(a02, b02_t, db1, sb1)

    a1s, b1s_t = _tc_tables([s_skip, s_02], cnt_m, W1s, b1s, np_src=NP1,
                            n_valid=N1, np_out=NP0, f=128, h=64, act=True)
    s_1s = _make_sc_conv(NP0, EP0, 32, 1)(a1s, b1s_t, db0, sb0)

    a11, b11_t = _tc_tables([s_skip, s_02], cnt_m, W11, b11, np_src=NP1,
                            n_valid=N1, np_out=NP1, f=128, h=32, act=True)
    s_11 = _make_sc_conv(NP1, EP1, 16, 2)(a11, b11_t, db1, sb1)

    a12, b12_t = _tc_tables([s_11], cnt_m, W12, b12, np_src=NP1, n_valid=N1,
                            np_out=NP0, f=32, h=64, act=False)
    s_12 = _make_sc_conv(NP0, EP0, 32, 1)(a12, b12_t, db0, sb0)

    afin, bfin_t = _tc_tables([s_1s, s_12], cnt_f, W_fin, b_fin, np_src=NP0,
                              n_valid=N0, np_out=NP0, f=64, h=64, act=True)
    s_fin = _make_sc_conv(NP0, EP0, 32, 1)(afin, bfin_t, db0, sb0)

    out = _tc_decoder(s_fin, cnt_f, W_d1, b_d1, W_d2, b_d2, gamma, beta)
    return out[:N0]


# speedup vs baseline: 4.6266x; 1.0172x over previous
import functools

import jax
import jax.numpy as jnp
from jax import lax
from jax.experimental import pallas as pl
from jax.experimental.pallas import tpu as pltpu
from jax.experimental.pallas import tpu_sc as plsc

N0, N1, N2 = 50000, 12500, 3125
NP0, NP1, NP2 = 50176, 13312, 4096
EP0, EP1, EP2 = 802816, 212992, 65536
LATENT = 32

_NC, _NS = 2, 16
_RZ = 32


def _dot(a, b, prec=lax.Precision.HIGHEST):
    return jax.lax.dot_general(a, b, (((1,), (0,)), ((), ())),
                               precision=prec,
                               preferred_element_type=jnp.float32)


def _bf(x):
    return x.astype(jnp.bfloat16).astype(jnp.float32)


def _leaky(x):
    return jnp.maximum(x, 0.01 * x)


def _sc_mesh():
    return plsc.VectorSubcoreMesh(
        core_axis_name="c", subcore_axis_name="s",
        num_cores=_NC, num_subcores=_NS)


@functools.lru_cache(maxsize=None)
def _make_sc_conv(npad, ep, hc, g):
    rows = ep // 128
    rows_w = rows // _NS
    n_chunks = rows_w // g
    rps = npad // _NS
    assert rows_w % g == 0 and rps % _RZ == 0 and hc % 16 == 0

    assert n_chunks % 2 == 0 and n_chunks >= 4

    def body(a_hbm, b_hbm, dst_hbm, src_hbm, out_hbm,
             accum, dbuf, sbuf, gd, gs, arows, brows, zbuf,
             lsem, gsemA, gsemB, ssem, zsem):
        c = lax.axis_index("c")
        s = lax.axis_index("s")
        gsems = (gsemA, gsemB)

        zv = jnp.zeros((16,), jnp.float32)

        def zfill(i, _):
            for j in range(hc // 16):
                zbuf[i, pl.ds(j * 16, 16)] = zv
            return 0
        lax.fori_loop(0, _RZ, zfill, 0)

        def zacc(k, _):
            pltpu.async_copy(zbuf, accum.at[pl.ds(s * rps + k * _RZ, _RZ)],
                             zsem)
            return 0
        lax.fori_loop(0, rps // _RZ, zacc, 0)

        def zdrain(k, _):
            pltpu.make_async_copy(
                zbuf, accum.at[pl.ds(s * rps + k * _RZ, _RZ)], zsem).wait()
            return 0
        lax.fori_loop(0, rps // _RZ, zdrain, 0)
        plsc.subcore_barrier()

        off = c * npad
        row0 = s * rows_w

        def idx_fire(t, sl):
            base = row0 + t * g
            pltpu.async_copy(dst_hbm.at[pl.ds(base, g)], dbuf.at[sl], lsem)
            pltpu.async_copy(src_hbm.at[pl.ds(base, g)], sbuf.at[sl], lsem)

        def idx_wait(t, sl):
            base = row0 + t * g
            pltpu.make_async_copy(
                dst_hbm.at[pl.ds(base, g)], dbuf.at[sl], lsem).wait()
            pltpu.make_async_copy(
                src_hbm.at[pl.ds(base, g)], sbuf.at[sl], lsem).wait()

        def shift_fire_gather(sl):
            for q in range(g):
                for j in range(8):
                    s16 = pl.ds(j * 16, 16)
                    gd[sl, q, s16] = dbuf[sl, q, s16] + off
                    gs[sl, q, s16] = sbuf[sl, q, s16] + off
            for q in range(g):
                pltpu.async_copy(
                    a_hbm.at[gd.at[sl, q]],
                    arows.at[sl, pl.ds(q * 128, 128)], gsems[sl])
                pltpu.async_copy(
                    b_hbm.at[gs.at[sl, q]],
                    brows.at[sl, pl.ds(q * 128, 128)], gsems[sl])

        def gather_wait(sl):
            for q in range(g):
                pltpu.make_async_copy(
                    a_hbm.at[gd.at[sl, q]],
                    arows.at[sl, pl.ds(q * 128, 128)], gsems[sl]).wait()
                pltpu.make_async_copy(
                    b_hbm.at[gs.at[sl, q]],
                    brows.at[sl, pl.ds(q * 128, 128)], gsems[sl]).wait()

        def leak(sl):
            def lk(r, _):
                for u in range(4):
                    for j in range(hc // 16):
                        s16 = pl.ds(j * 16, 16)
                        m = (arows[sl, r * 4 + u, s16]
                             + brows[sl, r * 4 + u, s16])
                        arows[sl, r * 4 + u, s16] = jnp.maximum(m, m * 0.01)
                return 0
            lax.fori_loop(0, g * 32, lk, 0)

        def scatter(sl):
            cps = []
            for q in range(g):
                cps.append(pltpu.async_copy(
                    arows.at[sl, pl.ds(q * 128, 128)],
                    accum.at[dbuf.at[sl, q]], ssem, add=True))
            for cp in cps:
                cp.wait()

        idx_fire(0, 0)
        idx_wait(0, 0)
        shift_fire_gather(0)
        idx_fire(1, 1)

        def pairstep(tt, _):
            for ph in range(2):
                t = 2 * tt + ph
                sl = ph
                osl = 1 - ph
                gather_wait(sl)
                leak(sl)
                scatter(sl)

                @pl.when(t + 2 < n_chunks)
                def _():
                    idx_fire(t + 2, sl)

                @pl.when(t + 1 < n_chunks)
                def _():
                    idx_wait(t + 1, osl)
                    shift_fire_gather(osl)
            return 0
        lax.fori_loop(0, n_chunks // 2, pairstep, 0)
        plsc.subcore_barrier()

        pltpu.sync_copy(accum.at[pl.ds(s * rps, rps)],
                        out_hbm.at[pl.ds(off + s * rps, rps)])

    return pl.kernel(
        body,
        out_type=jax.ShapeDtypeStruct((2 * npad, hc), jnp.float32),
        mesh=_sc_mesh(),
        compiler_params=pltpu.CompilerParams(use_tc_tiling_on_sc=False),
        scratch_types=[
            pltpu.VMEM_SHARED((npad, hc), jnp.float32),
            pltpu.VMEM((2, g, 128), jnp.int32),
            pltpu.VMEM((2, g, 128), jnp.int32),
            pltpu.VMEM((2, g, 128), jnp.int32),
            pltpu.VMEM((2, g, 128), jnp.int32),
            pltpu.VMEM((2, g * 128, hc), jnp.float32),
            pltpu.VMEM((2, g * 128, hc), jnp.float32),
            pltpu.VMEM((_RZ, hc), jnp.float32),
            pltpu.SemaphoreType.DMA,
            pltpu.SemaphoreType.DMA,
            pltpu.SemaphoreType.DMA,
            pltpu.SemaphoreType.DMA,
            pltpu.SemaphoreType.DMA,
        ],
    )


@functools.lru_cache(maxsize=None)
def _make_sc_conv_pair(npad, ep, hc, ride=False):
    hcw = hc + 16 if ride else hc
    rows = ep // 128
    rows_w = rows // _NS
    n_chunks = rows_w
    rps = npad // _NS
    nzfull, nzrem = divmod(rps, 128)
    assert n_chunks % 2 == 0 and n_chunks >= 4 and hc % 16 == 0

    def body(a1_hbm, b1_hbm, a2_hbm, b2_hbm, dst_hbm, src_hbm, out_hbm,
             accum, dbuf, sbuf, gd, gs, a1r, b1r, a2r, b2r, mbuf,
             lsem, gsem, ssemA, ssemB, zsem):
        c = lax.axis_index("c")
        s = lax.axis_index("s")
        ssems = (ssemA, ssemB)

        zv = jnp.zeros((16,), jnp.float32)

        def zfill(i, _):
            for j in range(hcw // 16):
                mbuf[0, i, pl.ds(j * 16, 16)] = zv
            return 0
        lax.fori_loop(0, 128, zfill, 0)

        def zacc(k, _):
            pltpu.async_copy(
                mbuf.at[0], accum.at[pl.ds(s * rps + k * 128, 128)], zsem)
            return 0
        lax.fori_loop(0, nzfull, zacc, 0)
        if nzrem:
            pltpu.async_copy(
                mbuf.at[0, pl.ds(0, nzrem)],
                accum.at[pl.ds(s * rps + nzfull * 128, nzrem)], zsem)

        def zdrain(k, _):
            pltpu.make_async_copy(
                mbuf.at[0], accum.at[pl.ds(s * rps + k * 128, 128)],
                zsem).wait()
            return 0
        lax.fori_loop(0, nzfull, zdrain, 0)
        if nzrem:
            pltpu.make_async_copy(
                mbuf.at[0, pl.ds(0, nzrem)],
                accum.at[pl.ds(s * rps + nzfull * 128, nzrem)], zsem).wait()
        if ride:
            ov = jnp.ones((16,), jnp.float32)

            def ofill(i, _):
                mbuf[0, i, pl.ds(hc, 16)] = ov
                mbuf[1, i, pl.ds(hc, 16)] = ov
                return 0
            lax.fori_loop(0, 128, ofill, 0)
        plsc.subcore_barrier()

        off = c * npad
        row0 = s * rows_w

        def idx_fire(t, sl):
            base = row0 + t
            pltpu.async_copy(dst_hbm.at[pl.ds(base, 1)], dbuf.at[sl], lsem)
            pltpu.async_copy(src_hbm.at[pl.ds(base, 1)], sbuf.at[sl], lsem)

        def idx_wait(t, sl):
            base = row0 + t
            pltpu.make_async_copy(
                dst_hbm.at[pl.ds(base, 1)], dbuf.at[sl], lsem).wait()
            pltpu.make_async_copy(
                src_hbm.at[pl.ds(base, 1)], sbuf.at[sl], lsem).wait()

        def drain_scatter(sl):
            pltpu.make_async_copy(
                mbuf.at[sl], accum.at[dbuf.at[sl, 0]], ssems[sl]).wait()

        def work(t, sl, osl, first):
            idx_wait(t, sl)
            for j in range(8):
                s16 = pl.ds(j * 16, 16)
                gd[0, s16] = dbuf[sl, 0, s16] + off
                gs[0, s16] = sbuf[sl, 0, s16] + off
            cps = [
                pltpu.async_copy(a1_hbm.at[gd.at[0]], a1r, gsem),
                pltpu.async_copy(b1_hbm.at[gs.at[0]], b1r, gsem),
                pltpu.async_copy(a2_hbm.at[gd.at[0]], a2r, gsem),
                pltpu.async_copy(b2_hbm.at[gs.at[0]], b2r, gsem),
            ]
            if not first:
                drain_scatter(osl)

            @pl.when(t + 1 < n_chunks)
            def _():
                idx_fire(t + 1, osl)
            for cp in cps:
                cp.wait()

            def lk(r, _):
                for u in range(4):
                    for j in range(hc // 16):
                        s16 = pl.ds(j * 16, 16)
                        m1 = a1r[r * 4 + u, s16] + b1r[r * 4 + u, s16]
                        m2 = a2r[r * 4 + u, s16] + b2r[r * 4 + u, s16]
                        mbuf[sl, r * 4 + u, s16] = (
                            jnp.maximum(m1, m1 * 0.01)
                            + jnp.maximum(m2, m2 * 0.01))
                return 0
            lax.fori_loop(0, 32, lk, 0)
            pltpu.async_copy(
                mbuf.at[sl], accum.at[dbuf.at[sl, 0]], ssems[sl], add=True)

        idx_fire(0, 0)
        work(0, 0, 1, True)
        work(1, 1, 0, False)

        def pairstep(tt, _):
            work(2 * tt + 2, 0, 1, False)
            work(2 * tt + 3, 1, 0, False)
            return 0
        lax.fori_loop(0, (n_chunks - 2) // 2, pairstep, 0)
        drain_scatter(1)
        plsc.subcore_barrier()

        pltpu.sync_copy(accum.at[pl.ds(s * rps, rps)],
                        out_hbm.at[pl.ds(off + s * rps, rps)])

    return pl.kernel(
        body,
        out_type=jax.ShapeDtypeStruct((2 * npad, hcw), jnp.float32),
        mesh=_sc_mesh(),
        compiler_params=pltpu.CompilerParams(use_tc_tiling_on_sc=False),
        scratch_types=[
            pltpu.VMEM_SHARED((npad, hcw), jnp.float32),
            pltpu.VMEM((2, 1, 128), jnp.int32),
            pltpu.VMEM((2, 1, 128), jnp.int32),
            pltpu.VMEM((1, 128), jnp.int32),
            pltpu.VMEM((1, 128), jnp.int32),
            pltpu.VMEM((128, hc), jnp.float32),
            pltpu.VMEM((128, hc), jnp.float32),
            pltpu.VMEM((128, hc), jnp.float32),
            pltpu.VMEM((128, hc), jnp.float32),
            pltpu.VMEM((2, 128, hcw), jnp.float32),
            pltpu.SemaphoreType.DMA,
            pltpu.SemaphoreType.DMA,
            pltpu.SemaphoreType.DMA,
            pltpu.SemaphoreType.DMA,
            pltpu.SemaphoreType.DMA,
        ],
    )


@functools.lru_cache(maxsize=None)
def _make_sc_count(npad, ep, g):
    rows = ep // 128
    rows_w = rows // (_NC * _NS)
    n_chunks = rows_w // g
    rps = npad // _NS
    assert rows_w % g == 0 and rps % _RZ == 0

    def body(dst_hbm, out_hbm, accum, dbuf, ones, zbuf, lsem, ssem):
        c = lax.axis_index("c")
        s = lax.axis_index("s")

        zv = jnp.zeros((16,), jnp.float32)
        ov = jnp.ones((16,), jnp.float32)

        def zfill(i, _):
            zbuf[i, pl.ds(0, 16)] = zv
            return 0
        lax.fori_loop(0, _RZ, zfill, 0)

        def ofill(i, _):
            ones[i, pl.ds(0, 16)] = ov
            return 0
        lax.fori_loop(0, 128, ofill, 0)

        def zacc(k, _):
            pltpu.async_copy(zbuf, accum.at[pl.ds(s * rps + k * _RZ, _RZ)],
                             ssem)
            return 0
        lax.fori_loop(0, rps // _RZ, zacc, 0)

        def zdrain(k, _):
            pltpu.make_async_copy(
                zbuf, accum.at[pl.ds(s * rps + k * _RZ, _RZ)], ssem).wait()
            return 0
        lax.fori_loop(0, rps // _RZ, zdrain, 0)
        plsc.subcore_barrier()

        row0 = (c * _NS + s) * rows_w

        def chunk(t, _):
            pltpu.async_copy(
                dst_hbm.at[pl.ds(row0 + t * g, g)], dbuf, lsem).wait()
            cps = []
            for q in range(g):
                cps.append(pltpu.async_copy(
                    ones, accum.at[dbuf.at[q]], ssem, add=True))
            for cp in cps:
                cp.wait()
            return 0
        lax.fori_loop(0, n_chunks, chunk, 0)
        plsc.subcore_barrier()

        pltpu.sync_copy(accum.at[pl.ds(s * rps, rps)],
                        out_hbm.at[pl.ds(c * npad + s * rps, rps)])

    return pl.kernel(
        body,
        out_type=jax.ShapeDtypeStruct((2 * npad, 16), jnp.float32),
        mesh=_sc_mesh(),
        compiler_params=pltpu.CompilerParams(use_tc_tiling_on_sc=False),
        scratch_types=[
            pltpu.VMEM_SHARED((npad, 16), jnp.float32),
            pltpu.VMEM((g, 128), jnp.int32),
            pltpu.VMEM((128, 16), jnp.float32),
            pltpu.VMEM((_RZ, 16), jnp.float32),
            pltpu.SemaphoreType.DMA,
            pltpu.SemaphoreType.DMA,
        ],
    )


def _tc_tables(sources, cnt, w, b, *, np_src, n_valid, np_out, f, h, act,
               cnt_full=False, fw=None):
    hc = h // 2
    hw = f // 2
    r = 1024
    nb = np_out // r
    nbs = np_src // r
    n_src = len(sources)
    wt = jnp.transpose(w.reshape(2 * f, 2, hc), (1, 0, 2))
    bt = b.reshape(2, 1, hc)
    fws = [fw or s_arr.shape[1] for s_arr in sources]
    n_cnt = 1 if cnt_full else 2
    cblk = cnt.shape[1] // 16 - 1

    def body(*refs):
        s_refs = refs[:2 * n_src]
        c_refs = refs[2 * n_src:2 * n_src + n_cnt]
        w_ref, b_ref, a_ref, b_out_ref = refs[2 * n_src + n_cnt:]
        i = pl.program_id(1)
        xa = s_refs[0][...][:, :fws[0]]
        xb = s_refs[1][...][:, :fws[0]]
        for k in range(1, n_src):
            xa = xa + s_refs[2 * k][...][:, :fws[k]]
            xb = xb + s_refs[2 * k + 1][...][:, :fws[k]]
        if cnt_full:
            cntv = jnp.maximum(c_refs[0][:, 16 * cblk], 1.0)[:, None]
        else:
            cntv = jnp.maximum(c_refs[0][:, 0] + c_refs[1][:, 0], 1.0)[:, None]
        xa = xa / cntv
        xb = xb / cntv
        if act:
            xa = _leaky(xa)
            xb = _leaky(xb)
        gid = i * r + lax.broadcasted_iota(jnp.int32, (r, 1), 0)
        valid = gid < n_valid
        xa = jnp.where(valid, xa, 0.0)
        xb = jnp.where(valid, xb, 0.0)
        wc = w_ref[0]
        wt = _bf(wc[:f, :])
        wj = _bf(wc[f:, :])
        xab = _bf(xa)
        xbb = _bf(xb)
        qa = _dot(xa, wj[:hw, :]) + _dot(xb, wj[hw:, :])
        pa = _dot(xab, wt[:hw, :]) + _dot(xbb, wt[hw:, :])
        a_ref[...] = (pa - qa) + b_ref[0]
        b_out_ref[...] = qa

    in_specs = []
    operands = []
    for s_arr in sources:
        hcs = s_arr.shape[1]
        in_specs += [
            pl.BlockSpec((r, hcs),
                         lambda c, i: (jnp.minimum(i, nbs - 1), 0)),
            pl.BlockSpec((r, hcs),
                         lambda c, i: (nbs + jnp.minimum(i, nbs - 1), 0)),
        ]
        operands += [s_arr, s_arr]
    if cnt_full:
        in_specs.append(pl.BlockSpec(
            (r, cnt.shape[1]), lambda c, i: (jnp.minimum(i, nbs - 1), 0)))
        operands.append(cnt)
    else:
        in_specs += [
            pl.BlockSpec((r, 16), lambda c, i: (jnp.minimum(i, nbs - 1), 0)),
            pl.BlockSpec(
                (r, 16), lambda c, i: (nbs + jnp.minimum(i, nbs - 1), 0)),
        ]
        operands += [cnt, cnt]
    in_specs += [
        pl.BlockSpec((1, 2 * f, hc), lambda c, i: (c, 0, 0)),
        pl.BlockSpec((1, 1, hc), lambda c, i: (c, 0, 0)),
    ]
    operands += [wt, bt]
    out_spec = pl.BlockSpec((r, hc), lambda c, i: (c * nb + i, 0))
    return pl.pallas_call(
        body,
        grid=(2, nb),
        in_specs=in_specs,
        out_specs=[out_spec, out_spec],
        out_shape=[jax.ShapeDtypeStruct((2 * np_out, hc), jnp.float32)] * 2,
    )(*operands)


def _tc_latent(z, w_up1, b_up1, w_up2p, b_up2p, w_bot, b_bot):
    r = 1024
    nb = NP2 // r
    hc = 128

    def body(z_ref, wu1_ref, bu1_ref, wu2_ref, bu2_ref, wb_ref, bb_ref,
             a_ref, b_out_ref):
        z2 = z_ref[0]
        hh = _leaky(_dot(z2, wu1_ref[...]) + bu1_ref[...][None, :])
        x0t = lax.dot_general(
            hh, wu2_ref[...], (((1,), (0,)), ((), ())),
            preferred_element_type=jnp.float32)
        x0 = jnp.transpose(x0t, (1, 0)) + bu2_ref[...][:, None]
        wt = _bf(wb_ref[:LATENT, :])
        wj = _bf(wb_ref[LATENT:, :])
        q = _dot(x0, wj)
        p = _dot(_bf(x0), wt)
        a_ref[...] = (p - q) + bb_ref[...][None, :]
        b_out_ref[...] = q

    out_spec = pl.BlockSpec((r, hc), lambda c, i: (c * nb + i, 0))
    return pl.pallas_call(
        body,
        grid=(2, nb),
        in_specs=[
            pl.BlockSpec((1, LATENT, 1), lambda c, i: (0, 0, 0)),
            pl.BlockSpec((1, 64), lambda c, i: (0, 0)),
            pl.BlockSpec((64,), lambda c, i: (0,)),
            pl.BlockSpec((64, r), lambda c, i: (0, i)),
            pl.BlockSpec((r,), lambda c, i: (i,)),
            pl.BlockSpec((2 * LATENT, hc), lambda c, i: (0, c)),
            pl.BlockSpec((hc,), lambda c, i: (c,)),
        ],
        out_specs=[out_spec, out_spec],
        out_shape=[jax.ShapeDtypeStruct((2 * NP2, hc), jnp.float32)] * 2,
    )(z, w_up1, b_up1, w_up2p, b_up2p, w_bot, b_bot)


def _tc_decoder(s_fin, cnt, w_d1, b_d1, w_d2, b_d2, gamma, beta):
    r = 1024
    nb = NP0 // r

    def body(sl_ref, sh_ref, cl_ref, ch_ref, w1_ref, b1_ref, w2_ref, b2_ref,
             g_ref, be_ref, o_ref):
        cntv = jnp.maximum(cl_ref[:, 0] + ch_ref[:, 0], 1.0)[:, None]
        xa = sl_ref[...] / cntv
        xb = sh_ref[...] / cntv
        hh = _leaky(_dot(xa, w1_ref[:32, :], None) + _dot(xb, w1_ref[32:, :], None)
                    + b1_ref[...][None, :])
        o = _dot(hh, w2_ref[...], None) + b2_ref[...][None, :]
        mu = jnp.mean(o, axis=-1, keepdims=True)
        d = o - mu
        var = jnp.mean(d * d, axis=-1, keepdims=True)
        o_ref[...] = (d / jnp.sqrt(var + 1e-5)) * g_ref[...][None, :] \
            + be_ref[...][None, :]

    return pl.pallas_call(
        body,
        grid=(nb,),
        in_specs=[
            pl.BlockSpec((r, 32), lambda i: (i, 0)),
            pl.BlockSpec((r, 32), lambda i: (nb + i, 0)),
            pl.BlockSpec((r, 16), lambda i: (i, 0)),
            pl.BlockSpec((r, 16), lambda i: (nb + i, 0)),
            pl.BlockSpec((64, 32), lambda i: (0, 0)),
            pl.BlockSpec((32,), lambda i: (0,)),
            pl.BlockSpec((32, 3), lambda i: (0, 0)),
            pl.BlockSpec((3,), lambda i: (0,)),
            pl.BlockSpec((3,), lambda i: (0,)),
            pl.BlockSpec((3,), lambda i: (0,)),
        ],
        out_specs=pl.BlockSpec((r, 3), lambda i: (i, 0)),
        out_shape=jax.ShapeDtypeStruct((NP0, 3), jnp.float32),
    )(s_fin, s_fin, cnt, cnt, w_d1, b_d1, w_d2, b_d2, gamma, beta)


def _prep_edges(edge_index, npad, ep):
    src = edge_index[0]
    dst = edge_index[1]
    pad = ep - src.shape[0]
    dummy = jnp.full((pad,), npad - 1, jnp.int32)
    src = jnp.concatenate([src, dummy]).reshape(ep // 128, 128)
    dst = jnp.concatenate([dst, dummy]).reshape(ep // 128, 128)
    return src, dst


def kernel(z, edge_index_bottom, edge_index_mid, edge_index_full, idx_mid,
           idx_full, W_up1, b_up1, W_up2, b_up2, W_bot, b_bot, W0s, b0s, W01,
           b01, W02, b02, W1s, b1s, W11, b11, W12, b12, W_fin, b_fin, W_d1,
           b_d1, W_d2, b_d2, gamma, beta):
    del idx_mid, idx_full
    sb2, db2 = _prep_edges(edge_index_bottom, NP2, EP2)
    sb1, db1 = _prep_edges(edge_index_mid, NP1, EP1)
    sb0, db0 = _prep_edges(edge_index_full, NP0, EP0)

    cnt_b = _make_sc_count(NP2, EP2, 2)(db2)
    cnt_f = _make_sc_count(NP0, EP0, 14)(db0)

    w_up2p = jnp.pad(W_up2, ((0, 0), (0, NP2 - N2)))
    b_up2p = jnp.pad(b_up2, (0, NP2 - N2))

    a1, b1 = _tc_latent(z, W_up1, b_up1, w_up2p, b_up2p, W_bot, b_bot)
    s1 = _make_sc_conv(NP2, EP2, 128, 1)(a1, b1, db2, sb2)

    a01, b01_t = _tc_tables([s1], cnt_b, W01, b01, np_src=NP2, n_valid=N2,
                            np_out=NP2, f=256, h=64, act=False)
    s_h = _make_sc_conv(NP2, EP2, 32, 2)(a01, b01_t, db2, sb2)

    a0s, b0s_t = _tc_tables([s1], cnt_b, W0s, b0s, np_src=NP2, n_valid=N2,
                            np_out=NP1, f=256, h=128, act=False)
    a02, b02_t = _tc_tables([s_h], cnt_b, W02, b02, np_src=NP2, n_valid=N2,
                            np_out=NP1, f=64, h=128, act=False)
    s_mid = _make_sc_conv_pair(NP1, EP1, 64, True)(
        a0s, b0s_t, a02, b02_t, db1, sb1)

    a11, b11_t = _tc_tables([s_mid], s_mid, W11, b11, np_src=NP1,
                            n_valid=N1, np_out=NP1, f=128, h=32, act=True,
                            cnt_full=True, fw=64)
    s_11 = _make_sc_conv(NP1, EP1, 16, 2)(a11, b11_t, db1, sb1)

    a1s, b1s_t = _tc_tables([s_mid], s_mid, W1s, b1s, np_src=NP1,
                            n_valid=N1, np_out=NP0, f=128, h=64, act=True,
                            cnt_full=True, fw=64)
    a12, b12_t = _tc_tables([s_11], s_mid, W12, b12, np_src=NP1, n_valid=N1,
                            np_out=NP0, f=32, h=64, act=False, cnt_full=True)
    s_full = _make_sc_conv_pair(NP0, EP0, 32)(
        a1s, b1s_t, a12, b12_t, db0, sb0)

    afin, bfin_t = _tc_tables([s_full], cnt_f, W_fin, b_fin, np_src=NP0,
                              n_valid=N0, np_out=NP0, f=64, h=64, act=True)
    s_fin = _make_sc_conv(NP0, EP0, 32, 1)(afin, bfin_t, db0, sb0)

    out = _tc_decoder(s_fin, cnt_f, W_d1, b_d1, W_d2, b_d2, gamma, beta)
    return out[:N0]
